# Initial kernel scaffold; baseline (speedup 1.0000x reference)
import jax
import jax.numpy as jnp
from jax.experimental import pallas as pl


def kernel(weights, spacing_starts, spacing_ends, nears, fars, num_samples):
    raise NotImplementedError("write your pallas kernel here")


# speedup vs baseline: 7.0655x; 7.0655x over previous
import functools

import jax
import jax.numpy as jnp
from jax import lax
from jax.experimental import pallas as pl
from jax.experimental.pallas import tpu as pltpu
from jax.experimental.pallas import tpu_sc as plsc

_L = 16
_NC = 2
_NS = 16
_NW = _NC * _NS


@functools.partial(jax.jit, static_argnums=(5,))
def _run(w2, s2, elast, nvec, fvec, S):
    R = w2.shape[0] // S
    NB = S + 1
    NO = 2 * NB
    RW = R // _NW
    SB = 128
    NSB = RW // SB
    GPB = SB // _L

    mesh = plsc.VectorSubcoreMesh(core_axis_name="c", subcore_axis_name="s")

    @functools.partial(
        pl.kernel,
        out_type=jax.ShapeDtypeStruct((R * NO,), jnp.float32),
        mesh=mesh,
        compiler_params=pltpu.CompilerParams(needs_layout_passes=False),
        scratch_types=[
            pltpu.VMEM((SB * S,), jnp.float32),
            pltpu.VMEM((SB * S,), jnp.float32),
            pltpu.VMEM((SB,), jnp.float32),
            pltpu.VMEM((SB,), jnp.float32),
            pltpu.VMEM((SB,), jnp.float32),
            pltpu.VMEM((SB * NO,), jnp.float32),
            pltpu.VMEM((NB * _L,), jnp.float32),
            pltpu.VMEM((NB * _L,), jnp.float32),
            pltpu.VMEM(((NB + 1) * _L,), jnp.int32),
            pltpu.VMEM((NB * _L,), jnp.int32),
        ],
    )
    def body(w_hbm, s_hbm, e_hbm, n_hbm, f_hbm, out_hbm,
             wblk, sblk, eblk, nblk, fblk, outblk, cdf, ebins, hist, hist2):
        wid = lax.axis_index("s") * _NC + lax.axis_index("c")
        base = wid * RW
        lane = lax.iota(jnp.int32, _L)
        ones_i = jnp.ones((_L,), jnp.int32)
        zeros_i = jnp.zeros((_L,), jnp.int32)
        zeros_f = jnp.zeros((_L,), jnp.float32)

        def group(g, _):
            g16 = g * _L
            rows = g16 + lane
            rowsS = rows * S
            rowsO = rows * NO
            near = nblk[pl.ds(g16, _L)]
            far = fblk[pl.ds(g16, _L)]
            fmn = far - near
            e64 = eblk[pl.ds(g16, _L)]
            ebins[pl.ds(S * _L, _L)] = e64

            def pa(s, acc):
                ww = plsc.load_gather(wblk, [rowsS + s])
                ee = plsc.load_gather(sblk, [rowsS + s])
                acc = acc + (ww + 0.01)
                cdf[pl.ds((s + 1) * _L, _L)] = acc
                ebins[pl.ds(s * _L, _L)] = ee
                hist[pl.ds(s * _L, _L)] = zeros_i
                hist2[pl.ds(s * _L, _L)] = zeros_i
                return acc

            ws = lax.fori_loop(0, S, pa, zeros_f)
            hist[pl.ds(S * _L, _L)] = zeros_i
            hist[pl.ds((S + 1) * _L, _L)] = zeros_i
            hist2[pl.ds(S * _L, _L)] = zeros_i
            hist[pl.ds(0, _L)] = ones_i
            cdf[pl.ds(0, _L)] = zeros_f
            pad = jnp.maximum(0.0, 1e-5 - ws)
            pad64 = pad * (1.0 / S)
            inv = 1.0 / (ws + pad)

            def pb(k, carry):
                cum = cdf[pl.ds(k * _L, _L)]
                kf = k.astype(jnp.float32)
                c = jnp.minimum(1.0, (cum + kf * pad64) * inv)
                cdf[pl.ds(k * _L, _L)] = c
                y = c * NB - 0.5
                tr = y.astype(jnp.int32)
                m = jnp.where(y > tr.astype(jnp.float32), tr + 1, tr)
                m = jnp.clip(m, 0, NB)
                plsc.addupdate_scatter(hist, [m * _L + lane], ones_i)
                return carry

            lax.fori_loop(1, NB, pb, 0)

            def pc(j, acc):
                acc = acc + hist[pl.ds(j * _L, _L)]
                below = jnp.minimum(acc - 1, S)
                above = jnp.minimum(acc, S)
                bidx = below * _L + lane
                aidx = above * _L + lane
                c0 = plsc.load_gather(cdf, [bidx])
                c1 = plsc.load_gather(cdf, [aidx])
                e0 = plsc.load_gather(ebins, [bidx])
                e1 = plsc.load_gather(ebins, [aidx])
                u = (j.astype(jnp.float32) * 2.0 + 1.0) * (1.0 / NO)
                num = u - c0
                den = c1 - c0
                t = jnp.clip(num / den, 0.0, 1.0)
                t = jnp.where(den == 0.0, jnp.where(num > 0.0, 1.0, 0.0), t)
                bval = e0 + t * (e1 - e0)
                val = near + bval * fmn
                plsc.store_scatter(outblk, [rowsO + (below + (j + 1))], val)
                plsc.addupdate_scatter(hist2, [bidx], ones_i)
                return acc

            lax.fori_loop(0, NB, pc, zeros_i)

            def pd(i, acc2):
                ei = ebins[pl.ds(i * _L, _L)]
                val = near + ei * fmn
                plsc.store_scatter(outblk, [rowsO + (acc2 + i)], val)
                return acc2 + hist2[pl.ds(i * _L, _L)]

            lax.fori_loop(0, NB, pd, zeros_i)
            return 0

        def superblock(sb, _):
            row0 = base + sb * SB
            pltpu.sync_copy(w_hbm.at[pl.ds(row0 * S, SB * S)], wblk)
            pltpu.sync_copy(s_hbm.at[pl.ds(row0 * S, SB * S)], sblk)
            pltpu.sync_copy(e_hbm.at[pl.ds(row0, SB)], eblk)
            pltpu.sync_copy(n_hbm.at[pl.ds(row0, SB)], nblk)
            pltpu.sync_copy(f_hbm.at[pl.ds(row0, SB)], fblk)
            lax.fori_loop(0, GPB, group, 0)
            pltpu.sync_copy(outblk, out_hbm.at[pl.ds(row0 * NO, SB * NO)])
            return 0

        lax.fori_loop(0, NSB, superblock, 0)

    return body(w2, s2, elast, nvec, fvec).reshape(R, NO)


def kernel(weights, spacing_starts, spacing_ends, nears, fars, num_samples=64):
    R, S = weights.shape[0], weights.shape[1]
    w2 = weights.reshape(R * S)
    s2 = spacing_starts.reshape(R * S)
    elast = spacing_ends[:, -1, 0]
    return _run(w2, s2, elast, nears.reshape(R), fars.reshape(R), S)


# speedup vs baseline: 7.3968x; 1.0469x over previous
import functools

import jax
import jax.numpy as jnp
from jax import lax
from jax.experimental import pallas as pl
from jax.experimental.pallas import tpu as pltpu
from jax.experimental.pallas import tpu_sc as plsc

_L = 16
_NC = 2
_NS = 16
_NW = _NC * _NS


@functools.partial(jax.jit, static_argnums=(5,))
def _run(w2, s2, elast, nvec, fvec, S):
    R = w2.shape[0] // S
    NB = S + 1
    NO = 2 * NB
    RW = R // _NW
    SB = 128
    NSB = RW // SB
    GPB = SB // _L
    NBL = NB * _L

    mesh = plsc.VectorSubcoreMesh(core_axis_name="c", subcore_axis_name="s")

    @functools.partial(
        pl.kernel,
        out_type=jax.ShapeDtypeStruct((R * NO,), jnp.float32),
        mesh=mesh,
        compiler_params=pltpu.CompilerParams(needs_layout_passes=False),
        scratch_types=[
            pltpu.VMEM((SB * S,), jnp.float32),
            pltpu.VMEM((SB * S,), jnp.float32),
            pltpu.VMEM((SB,), jnp.float32),
            pltpu.VMEM((SB,), jnp.float32),
            pltpu.VMEM((SB,), jnp.float32),
            pltpu.VMEM((SB * NO,), jnp.float32),
            pltpu.VMEM((2 * NBL,), jnp.float32),
            pltpu.VMEM((2 * NBL,), jnp.float32),
            pltpu.VMEM((2 * (NB + 1) * _L,), jnp.int32),
        ],
    )
    def body(w_hbm, s_hbm, e_hbm, n_hbm, f_hbm, out_hbm,
             wblk, sblk, eblk, nblk, fblk, outblk, cdf, ebins, hist):
        wid = lax.axis_index("s") * _NC + lax.axis_index("c")
        base = wid * RW
        lane = lax.iota(jnp.int32, _L)
        ones_i = jnp.ones((_L,), jnp.int32)
        zeros_i = jnp.zeros((_L,), jnp.int32)
        zeros_f = jnp.zeros((_L,), jnp.float32)
        HL = (NB + 1) * _L

        def gpair(gp, _):
            st = []
            for t in (0, 1):
                g16 = (gp * 2 + t) * _L
                rows = g16 + lane
                near = nblk[pl.ds(g16, _L)]
                far = fblk[pl.ds(g16, _L)]
                st.append(dict(
                    g16=g16, rows=rows, rowsS=rows * S, rowsO=rows * NO,
                    near=near, fmn=far - near, co=t * NBL, ho=t * HL,
                ))

            def pa(s, accs):
                out = []
                for t in (0, 1):
                    d = st[t]
                    ww = plsc.load_gather(wblk, [d["rowsS"] + s])
                    ee = plsc.load_gather(sblk, [d["rowsS"] + s])
                    acc = accs[t] + (ww + 0.01)
                    cdf[pl.ds(d["co"] + (s + 1) * _L, _L)] = acc
                    ebins[pl.ds(d["co"] + s * _L, _L)] = ee
                    hist[pl.ds(d["ho"] + s * _L, _L)] = zeros_i
                    out.append(acc)
                return tuple(out)

            wss = lax.fori_loop(0, S, pa, (zeros_f, zeros_f))
            pads = []
            for t in (0, 1):
                d = st[t]
                hist[pl.ds(d["ho"] + S * _L, _L)] = zeros_i
                hist[pl.ds(d["ho"] + (S + 1) * _L, _L)] = zeros_i
                cdf[pl.ds(d["co"], _L)] = zeros_f
                ebins[pl.ds(d["co"] + S * _L, _L)] = eblk[pl.ds(d["g16"], _L)]
                ws = wss[t]
                pad = jnp.maximum(0.0, 1e-5 - ws)
                pads.append((pad * (1.0 / S), 1.0 / (ws + pad)))

            def pb(k, carry):
                kf = k.astype(jnp.float32)
                for t in (0, 1):
                    d = st[t]
                    pad64, inv = pads[t]
                    cum = cdf[pl.ds(d["co"] + k * _L, _L)]
                    c = jnp.minimum(1.0, (cum + kf * pad64) * inv)
                    cdf[pl.ds(d["co"] + k * _L, _L)] = c
                    y = c * NB - 0.5
                    tr = y.astype(jnp.int32)
                    m = jnp.where(y > tr.astype(jnp.float32), tr + 1, tr)
                    plsc.addupdate_scatter(hist, [d["ho"] + m * _L + lane], ones_i)
                    ek = ebins[pl.ds(d["co"] + k * _L, _L)]
                    val = d["near"] + ek * d["fmn"]
                    plsc.store_scatter(outblk, [d["rowsO"] + (m + k)], val)
                return carry

            lax.fori_loop(0, NB, pb, 0)

            def pc(j, accs):
                u = (j.astype(jnp.float32) * 2.0 + 1.0) * (1.0 / NO)
                jp1 = j + 1
                out = []
                for t in (0, 1):
                    d = st[t]
                    acc = accs[t] + hist[pl.ds(d["ho"] + j * _L, _L)]
                    below = jnp.minimum(acc - 1, S)
                    above = jnp.minimum(acc, S)
                    bidx = below * _L + lane
                    aidx = above * _L + lane
                    c0 = plsc.load_gather(cdf, [d["co"] + bidx])
                    c1 = plsc.load_gather(cdf, [d["co"] + aidx])
                    e0 = plsc.load_gather(ebins, [d["co"] + bidx])
                    e1 = plsc.load_gather(ebins, [d["co"] + aidx])
                    num = u - c0
                    den = c1 - c0
                    tt = jnp.clip(num / den, 0.0, 1.0)
                    tt = jnp.where(den == 0.0, jnp.where(num > 0.0, 1.0, 0.0), tt)
                    bval = e0 + tt * (e1 - e0)
                    val = d["near"] + bval * d["fmn"]
                    plsc.store_scatter(outblk, [d["rowsO"] + (below + jp1)], val)
                    out.append(acc)
                return tuple(out)

            lax.fori_loop(0, NB, pc, (zeros_i, zeros_i))
            return 0

        def superblock(sb, _):
            row0 = base + sb * SB
            pltpu.sync_copy(w_hbm.at[pl.ds(row0 * S, SB * S)], wblk)
            pltpu.sync_copy(s_hbm.at[pl.ds(row0 * S, SB * S)], sblk)
            pltpu.sync_copy(e_hbm.at[pl.ds(row0, SB)], eblk)
            pltpu.sync_copy(n_hbm.at[pl.ds(row0, SB)], nblk)
            pltpu.sync_copy(f_hbm.at[pl.ds(row0, SB)], fblk)
            lax.fori_loop(0, GPB // 2, gpair, 0)
            pltpu.sync_copy(outblk, out_hbm.at[pl.ds(row0 * NO, SB * NO)])
            return 0

        lax.fori_loop(0, NSB, superblock, 0)

    return body(w2, s2, elast, nvec, fvec).reshape(R, NO)


def kernel(weights, spacing_starts, spacing_ends, nears, fars, num_samples=64):
    R, S = weights.shape[0], weights.shape[1]
    w2 = weights.reshape(R * S)
    s2 = spacing_starts.reshape(R * S)
    elast = spacing_ends[:, -1, 0]
    return _run(w2, s2, elast, nears.reshape(R), fars.reshape(R), S)


# speedup vs baseline: 7.4125x; 1.0021x over previous
import functools

import jax
import jax.numpy as jnp
from jax import lax
from jax.experimental import pallas as pl
from jax.experimental.pallas import tpu as pltpu
from jax.experimental.pallas import tpu_sc as plsc

_L = 16
_NC = 2
_NS = 16
_NW = _NC * _NS


@functools.partial(jax.jit, static_argnums=(5,))
def _run(w2, s2, elast, nvec, fvec, S):
    R = w2.shape[0] // S
    NB = S + 1
    NO = 2 * NB
    RW = R // _NW
    SB = 128
    NSB = RW // SB
    GPB = SB // _L
    NBL = NB * _L

    mesh = plsc.VectorSubcoreMesh(core_axis_name="c", subcore_axis_name="s")

    @functools.partial(
        pl.kernel,
        out_type=jax.ShapeDtypeStruct((R * NO,), jnp.float32),
        mesh=mesh,
        compiler_params=pltpu.CompilerParams(needs_layout_passes=False),
        scratch_types=[
            pltpu.VMEM((SB * S,), jnp.float32),
            pltpu.VMEM((SB * S,), jnp.float32),
            pltpu.VMEM((SB,), jnp.float32),
            pltpu.VMEM((SB,), jnp.float32),
            pltpu.VMEM((SB,), jnp.float32),
            pltpu.VMEM((SB * NO,), jnp.float32),
            pltpu.VMEM((NBL,), jnp.float32),
            pltpu.VMEM((NBL,), jnp.float32),
            pltpu.VMEM((NBL,), jnp.float32),
            pltpu.VMEM((NBL,), jnp.float32),
            pltpu.VMEM(((NB + 1) * _L,), jnp.int32),
            pltpu.VMEM(((NB + 1) * _L,), jnp.int32),
        ],
    )
    def body(w_hbm, s_hbm, e_hbm, n_hbm, f_hbm, out_hbm,
             wblk, sblk, eblk, nblk, fblk, outblk, cdf0, cdf1, eb0, eb1, h0, h1):
        cdfs, ebinss, hists = (cdf0, cdf1), (eb0, eb1), (h0, h1)
        wid = lax.axis_index("s") * _NC + lax.axis_index("c")
        base = wid * RW
        lane = lax.iota(jnp.int32, _L)
        ones_i = jnp.ones((_L,), jnp.int32)
        zeros_i = jnp.zeros((_L,), jnp.int32)
        zeros_f = jnp.zeros((_L,), jnp.float32)
        def gpair(gp, _):
            st = []
            for t in (0, 1):
                g16 = (gp * 2 + t) * _L
                rows = g16 + lane
                near = nblk[pl.ds(g16, _L)]
                far = fblk[pl.ds(g16, _L)]
                st.append(dict(
                    g16=g16, rows=rows, rowsS=rows * S, rowsO=rows * NO,
                    near=near, fmn=far - near,
                    cdf=cdfs[t], ebins=ebinss[t], hist=hists[t],
                ))

            def pa(s, accs):
                out = []
                for t in (0, 1):
                    d = st[t]
                    ww = plsc.load_gather(wblk, [d["rowsS"] + s])
                    ee = plsc.load_gather(sblk, [d["rowsS"] + s])
                    acc = accs[t] + (ww + 0.01)
                    d["cdf"][pl.ds((s + 1) * _L, _L)] = acc
                    d["ebins"][pl.ds(s * _L, _L)] = ee
                    d["hist"][pl.ds(s * _L, _L)] = zeros_i
                    out.append(acc)
                return tuple(out)

            wss = lax.fori_loop(0, S, pa, (zeros_f, zeros_f))
            pads = []
            for t in (0, 1):
                d = st[t]
                d["hist"][pl.ds(S * _L, _L)] = zeros_i
                d["hist"][pl.ds((S + 1) * _L, _L)] = zeros_i
                d["cdf"][pl.ds(0, _L)] = zeros_f
                d["ebins"][pl.ds(S * _L, _L)] = eblk[pl.ds(d["g16"], _L)]
                ws = wss[t]
                pad = jnp.maximum(0.0, 1e-5 - ws)
                pads.append((pad * (1.0 / S), 1.0 / (ws + pad)))

            def pb(k, carry):
                kf = k.astype(jnp.float32)
                for t in (0, 1):
                    d = st[t]
                    pad64, inv = pads[t]
                    cum = d["cdf"][pl.ds(k * _L, _L)]
                    c = jnp.minimum(1.0, (cum + kf * pad64) * inv)
                    d["cdf"][pl.ds(k * _L, _L)] = c
                    y = c * NB - 0.5
                    tr = y.astype(jnp.int32)
                    m = jnp.where(y > tr.astype(jnp.float32), tr + 1, tr)
                    plsc.addupdate_scatter(d["hist"], [m * _L + lane], ones_i)
                    ek = d["ebins"][pl.ds(k * _L, _L)]
                    val = d["near"] + ek * d["fmn"]
                    plsc.store_scatter(outblk, [d["rowsO"] + (m + k)], val)
                return carry

            lax.fori_loop(0, NB, pb, 0)

            def pc(j, accs):
                u = (j.astype(jnp.float32) * 2.0 + 1.0) * (1.0 / NO)
                jp1 = j + 1
                out = []
                for t in (0, 1):
                    d = st[t]
                    acc = accs[t] + d["hist"][pl.ds(j * _L, _L)]
                    below = jnp.minimum(acc - 1, S)
                    above = jnp.minimum(acc, S)
                    bidx = below * _L + lane
                    aidx = above * _L + lane
                    c0 = plsc.load_gather(d["cdf"], [bidx])
                    c1 = plsc.load_gather(d["cdf"], [aidx])
                    e0 = plsc.load_gather(d["ebins"], [bidx])
                    e1 = plsc.load_gather(d["ebins"], [aidx])
                    num = u - c0
                    den = c1 - c0
                    tt = jnp.clip(num / den, 0.0, 1.0)
                    tt = jnp.where(den == 0.0, jnp.where(num > 0.0, 1.0, 0.0), tt)
                    bval = e0 + tt * (e1 - e0)
                    val = d["near"] + bval * d["fmn"]
                    plsc.store_scatter(outblk, [d["rowsO"] + (below + jp1)], val)
                    out.append(acc)
                return tuple(out)

            lax.fori_loop(0, NB, pc, (zeros_i, zeros_i))
            return 0

        def superblock(sb, _):
            row0 = base + sb * SB
            pltpu.sync_copy(w_hbm.at[pl.ds(row0 * S, SB * S)], wblk)
            pltpu.sync_copy(s_hbm.at[pl.ds(row0 * S, SB * S)], sblk)
            pltpu.sync_copy(e_hbm.at[pl.ds(row0, SB)], eblk)
            pltpu.sync_copy(n_hbm.at[pl.ds(row0, SB)], nblk)
            pltpu.sync_copy(f_hbm.at[pl.ds(row0, SB)], fblk)
            lax.fori_loop(0, GPB // 2, gpair, 0)
            pltpu.sync_copy(outblk, out_hbm.at[pl.ds(row0 * NO, SB * NO)])
            return 0

        lax.fori_loop(0, NSB, superblock, 0)

    return body(w2, s2, elast, nvec, fvec).reshape(R, NO)


def kernel(weights, spacing_starts, spacing_ends, nears, fars, num_samples=64):
    R, S = weights.shape[0], weights.shape[1]
    w2 = weights.reshape(R * S)
    s2 = spacing_starts.reshape(R * S)
    elast = spacing_ends[:, -1, 0]
    return _run(w2, s2, elast, nears.reshape(R), fars.reshape(R), S)


# speedup vs baseline: 13.2639x; 1.7894x over previous
import functools

import jax
import jax.numpy as jnp
from jax import lax
from jax.experimental import pallas as pl
from jax.experimental.pallas import tpu as pltpu
from jax.experimental.pallas import tpu_sc as plsc

_L = 16
_NC = 2
_NS = 16
_NW = _NC * _NS


@functools.partial(jax.jit, static_argnums=(5,))
def _run(w2, s2, elast, nvec, fvec, S):
    R = w2.shape[0] // S
    NB = S + 1
    NO = 2 * NB
    RW = R // _NW
    SB = 128
    NSB = RW // SB
    GPB = SB // _L
    NBL = NB * _L

    mesh = plsc.VectorSubcoreMesh(core_axis_name="c", subcore_axis_name="s")

    @functools.partial(
        pl.kernel,
        out_type=jax.ShapeDtypeStruct((R * NO,), jnp.float32),
        mesh=mesh,
        compiler_params=pltpu.CompilerParams(needs_layout_passes=False),
        scratch_types=[
            pltpu.VMEM((SB * S,), jnp.float32),
            pltpu.VMEM((SB * S,), jnp.float32),
            pltpu.VMEM((SB,), jnp.float32),
            pltpu.VMEM((SB,), jnp.float32),
            pltpu.VMEM((SB,), jnp.float32),
            pltpu.VMEM((SB * NO,), jnp.float32),
            pltpu.VMEM((NBL,), jnp.float32),
            pltpu.VMEM((NBL,), jnp.float32),
            pltpu.VMEM((NBL,), jnp.float32),
            pltpu.VMEM((NBL,), jnp.float32),
            pltpu.VMEM(((NB + 1) * _L,), jnp.int32),
            pltpu.VMEM(((NB + 1) * _L,), jnp.int32),
        ],
    )
    def body(w_hbm, s_hbm, e_hbm, n_hbm, f_hbm, out_hbm,
             wblk, sblk, eblk, nblk, fblk, outblk, cdf0, cdf1, eb0, eb1, h0, h1):
        cdfs, ebinss, hists = (cdf0, cdf1), (eb0, eb1), (h0, h1)
        wid = lax.axis_index("s") * _NC + lax.axis_index("c")
        base = wid * RW
        lane = lax.iota(jnp.int32, _L)
        ones_i = jnp.ones((_L,), jnp.int32)
        zeros_i = jnp.zeros((_L,), jnp.int32)
        zeros_f = jnp.zeros((_L,), jnp.float32)
        def gpair(gp, _):
            st = []
            for t in (0, 1):
                g16 = (gp * 2 + t) * _L
                rows = g16 + lane
                near = nblk[pl.ds(g16, _L)]
                far = fblk[pl.ds(g16, _L)]
                st.append(dict(
                    g16=g16, rows=rows, rowsS=rows * S, rowsO=rows * NO,
                    near=near, fmn=far - near,
                    cdf=cdfs[t], ebins=ebinss[t], hist=hists[t],
                ))

            def pa(s, accs):
                out = []
                for t in (0, 1):
                    d = st[t]
                    ww = plsc.load_gather(wblk, [d["rowsS"] + s])
                    ee = plsc.load_gather(sblk, [d["rowsS"] + s])
                    acc = accs[t] + (ww + 0.01)
                    d["cdf"][pl.ds((s + 1) * _L, _L)] = acc
                    d["ebins"][pl.ds(s * _L, _L)] = ee
                    d["hist"][pl.ds(s * _L, _L)] = zeros_i
                    out.append(acc)
                return tuple(out)

            wss = plsc.parallel_loop(0, S, 1, unroll=2, carry=(zeros_f, zeros_f))(pa)
            pads = []
            for t in (0, 1):
                d = st[t]
                d["hist"][pl.ds(S * _L, _L)] = zeros_i
                d["hist"][pl.ds((S + 1) * _L, _L)] = zeros_i
                d["cdf"][pl.ds(0, _L)] = zeros_f
                d["ebins"][pl.ds(S * _L, _L)] = eblk[pl.ds(d["g16"], _L)]
                ws = wss[t]
                pad = jnp.maximum(0.0, 1e-5 - ws)
                pads.append((pad * (1.0 / S), 1.0 / (ws + pad)))

            def pb(k, carry):
                kf = k.astype(jnp.float32)
                for t in (0, 1):
                    d = st[t]
                    pad64, inv = pads[t]
                    cum = d["cdf"][pl.ds(k * _L, _L)]
                    c = jnp.minimum(1.0, (cum + kf * pad64) * inv)
                    d["cdf"][pl.ds(k * _L, _L)] = c
                    y = c * NB - 0.5
                    tr = y.astype(jnp.int32)
                    m = jnp.where(y > tr.astype(jnp.float32), tr + 1, tr)
                    plsc.addupdate_scatter(d["hist"], [m * _L + lane], ones_i)
                    ek = d["ebins"][pl.ds(k * _L, _L)]
                    val = d["near"] + ek * d["fmn"]
                    plsc.store_scatter(outblk, [d["rowsO"] + (m + k)], val)
                return carry

            plsc.parallel_loop(0, NB, 1, unroll=2, carry=jnp.int32(0))(pb)

            def pc(j, accs):
                u = (j.astype(jnp.float32) * 2.0 + 1.0) * (1.0 / NO)
                jp1 = j + 1
                out = []
                for t in (0, 1):
                    d = st[t]
                    acc = accs[t] + d["hist"][pl.ds(j * _L, _L)]
                    below = jnp.minimum(acc - 1, S)
                    above = jnp.minimum(acc, S)
                    bidx = below * _L + lane
                    aidx = above * _L + lane
                    c0 = plsc.load_gather(d["cdf"], [bidx])
                    c1 = plsc.load_gather(d["cdf"], [aidx])
                    e0 = plsc.load_gather(d["ebins"], [bidx])
                    e1 = plsc.load_gather(d["ebins"], [aidx])
                    num = u - c0
                    den = c1 - c0
                    tt = jnp.clip(num / den, 0.0, 1.0)
                    tt = jnp.where(den == 0.0, jnp.where(num > 0.0, 1.0, 0.0), tt)
                    bval = e0 + tt * (e1 - e0)
                    val = d["near"] + bval * d["fmn"]
                    plsc.store_scatter(outblk, [d["rowsO"] + (below + jp1)], val)
                    out.append(acc)
                return tuple(out)

            plsc.parallel_loop(0, NB, 1, unroll=2, carry=(zeros_i, zeros_i))(pc)
            return 0

        def superblock(sb, _):
            row0 = base + sb * SB
            pltpu.sync_copy(w_hbm.at[pl.ds(row0 * S, SB * S)], wblk)
            pltpu.sync_copy(s_hbm.at[pl.ds(row0 * S, SB * S)], sblk)
            pltpu.sync_copy(e_hbm.at[pl.ds(row0, SB)], eblk)
            pltpu.sync_copy(n_hbm.at[pl.ds(row0, SB)], nblk)
            pltpu.sync_copy(f_hbm.at[pl.ds(row0, SB)], fblk)
            lax.fori_loop(0, GPB // 2, gpair, 0)
            pltpu.sync_copy(outblk, out_hbm.at[pl.ds(row0 * NO, SB * NO)])
            return 0

        lax.fori_loop(0, NSB, superblock, 0)

    return body(w2, s2, elast, nvec, fvec).reshape(R, NO)


def kernel(weights, spacing_starts, spacing_ends, nears, fars, num_samples=64):
    R, S = weights.shape[0], weights.shape[1]
    w2 = weights.reshape(R * S)
    s2 = spacing_starts.reshape(R * S)
    elast = spacing_ends[:, -1, 0]
    return _run(w2, s2, elast, nears.reshape(R), fars.reshape(R), S)


# speedup vs baseline: 13.2675x; 1.0003x over previous
import functools

import jax
import jax.numpy as jnp
from jax import lax
from jax.experimental import pallas as pl
from jax.experimental.pallas import tpu as pltpu
from jax.experimental.pallas import tpu_sc as plsc

_L = 16
_NC = 2
_NS = 16
_NW = _NC * _NS


@functools.partial(jax.jit, static_argnums=(5,))
def _run(w2, s2, elast, nvec, fvec, S):
    R = w2.shape[0] // S
    NB = S + 1
    NO = 2 * NB
    RW = R // _NW
    SB = 128
    NSB = RW // SB
    GPB = SB // _L
    NBL = NB * _L

    mesh = plsc.VectorSubcoreMesh(core_axis_name="c", subcore_axis_name="s")

    @functools.partial(
        pl.kernel,
        out_type=jax.ShapeDtypeStruct((R * NO,), jnp.float32),
        mesh=mesh,
        compiler_params=pltpu.CompilerParams(needs_layout_passes=False),
        scratch_types=[
            pltpu.VMEM((SB * S,), jnp.float32),
            pltpu.VMEM((SB * S,), jnp.float32),
            pltpu.VMEM((SB,), jnp.float32),
            pltpu.VMEM((SB,), jnp.float32),
            pltpu.VMEM((SB,), jnp.float32),
            pltpu.VMEM((SB * NO,), jnp.float32),
            pltpu.VMEM((NBL,), jnp.float32),
            pltpu.VMEM((NBL,), jnp.float32),
            pltpu.VMEM((NBL,), jnp.float32),
            pltpu.VMEM((NBL,), jnp.float32),
            pltpu.VMEM(((NB + 1) * _L,), jnp.int32),
            pltpu.VMEM(((NB + 1) * _L,), jnp.int32),
        ],
    )
    def body(w_hbm, s_hbm, e_hbm, n_hbm, f_hbm, out_hbm,
             wblk, sblk, eblk, nblk, fblk, outblk, cdf0, cdf1, eb0, eb1, h0, h1):
        cdfs, ebinss, hists = (cdf0, cdf1), (eb0, eb1), (h0, h1)
        wid = lax.axis_index("s") * _NC + lax.axis_index("c")
        base = wid * RW
        lane = lax.iota(jnp.int32, _L)
        ones_i = jnp.ones((_L,), jnp.int32)
        zeros_i = jnp.zeros((_L,), jnp.int32)
        zeros_f = jnp.zeros((_L,), jnp.float32)
        def gpair(gp, _):
            st = []
            for t in (0, 1):
                g16 = (gp * 2 + t) * _L
                rows = g16 + lane
                near = nblk[pl.ds(g16, _L)]
                far = fblk[pl.ds(g16, _L)]
                st.append(dict(
                    g16=g16, rows=rows, rowsS=rows * S, rowsO=rows * NO,
                    near=near, fmn=far - near,
                    cdf=cdfs[t], ebins=ebinss[t], hist=hists[t],
                ))

            def pa(s, accs):
                out = []
                for t in (0, 1):
                    d = st[t]
                    ww = plsc.load_gather(wblk, [d["rowsS"] + s])
                    ee = plsc.load_gather(sblk, [d["rowsS"] + s])
                    acc = accs[t] + (ww + 0.01)
                    d["cdf"][pl.ds((s + 1) * _L, _L)] = acc
                    d["ebins"][pl.ds(s * _L, _L)] = ee
                    d["hist"][pl.ds(s * _L, _L)] = zeros_i
                    out.append(acc)
                return tuple(out)

            wss = plsc.parallel_loop(0, S, 1, unroll=4, carry=(zeros_f, zeros_f))(pa)
            pads = []
            for t in (0, 1):
                d = st[t]
                d["hist"][pl.ds(S * _L, _L)] = zeros_i
                d["hist"][pl.ds((S + 1) * _L, _L)] = zeros_i
                d["cdf"][pl.ds(0, _L)] = zeros_f
                d["ebins"][pl.ds(S * _L, _L)] = eblk[pl.ds(d["g16"], _L)]
                ws = wss[t]
                pad = jnp.maximum(0.0, 1e-5 - ws)
                pads.append((pad * (1.0 / S), 1.0 / (ws + pad)))

            def pb(k, carry):
                kf = k.astype(jnp.float32)
                for t in (0, 1):
                    d = st[t]
                    pad64, inv = pads[t]
                    cum = d["cdf"][pl.ds(k * _L, _L)]
                    c = jnp.minimum(1.0, (cum + kf * pad64) * inv)
                    d["cdf"][pl.ds(k * _L, _L)] = c
                    y = c * NB - 0.5
                    tr = y.astype(jnp.int32)
                    m = jnp.where(y > tr.astype(jnp.float32), tr + 1, tr)
                    plsc.addupdate_scatter(d["hist"], [m * _L + lane], ones_i)
                    ek = d["ebins"][pl.ds(k * _L, _L)]
                    val = d["near"] + ek * d["fmn"]
                    plsc.store_scatter(outblk, [d["rowsO"] + (m + k)], val)
                return carry

            plsc.parallel_loop(0, NB, 1, unroll=4, carry=jnp.int32(0))(pb)

            def pc(j, accs):
                u = (j.astype(jnp.float32) * 2.0 + 1.0) * (1.0 / NO)
                jp1 = j + 1
                out = []
                for t in (0, 1):
                    d = st[t]
                    acc = accs[t] + d["hist"][pl.ds(j * _L, _L)]
                    below = jnp.minimum(acc - 1, S)
                    above = jnp.minimum(acc, S)
                    bidx = below * _L + lane
                    aidx = above * _L + lane
                    c0 = plsc.load_gather(d["cdf"], [bidx])
                    c1 = plsc.load_gather(d["cdf"], [aidx])
                    e0 = plsc.load_gather(d["ebins"], [bidx])
                    e1 = plsc.load_gather(d["ebins"], [aidx])
                    num = u - c0
                    den = c1 - c0
                    tt = jnp.clip(num / den, 0.0, 1.0)
                    tt = jnp.where(den == 0.0, jnp.where(num > 0.0, 1.0, 0.0), tt)
                    bval = e0 + tt * (e1 - e0)
                    val = d["near"] + bval * d["fmn"]
                    plsc.store_scatter(outblk, [d["rowsO"] + (below + jp1)], val)
                    out.append(acc)
                return tuple(out)

            plsc.parallel_loop(0, NB, 1, unroll=4, carry=(zeros_i, zeros_i))(pc)
            return 0

        def superblock(sb, _):
            row0 = base + sb * SB
            pltpu.sync_copy(w_hbm.at[pl.ds(row0 * S, SB * S)], wblk)
            pltpu.sync_copy(s_hbm.at[pl.ds(row0 * S, SB * S)], sblk)
            pltpu.sync_copy(e_hbm.at[pl.ds(row0, SB)], eblk)
            pltpu.sync_copy(n_hbm.at[pl.ds(row0, SB)], nblk)
            pltpu.sync_copy(f_hbm.at[pl.ds(row0, SB)], fblk)
            lax.fori_loop(0, GPB // 2, gpair, 0)
            pltpu.sync_copy(outblk, out_hbm.at[pl.ds(row0 * NO, SB * NO)])
            return 0

        lax.fori_loop(0, NSB, superblock, 0)

    return body(w2, s2, elast, nvec, fvec).reshape(R, NO)


def kernel(weights, spacing_starts, spacing_ends, nears, fars, num_samples=64):
    R, S = weights.shape[0], weights.shape[1]
    w2 = weights.reshape(R * S)
    s2 = spacing_starts.reshape(R * S)
    elast = spacing_ends[:, -1, 0]
    return _run(w2, s2, elast, nears.reshape(R), fars.reshape(R), S)


# speedup vs baseline: 15.1826x; 1.1443x over previous
import functools

import jax
import jax.numpy as jnp
from jax import lax
from jax.experimental import pallas as pl
from jax.experimental.pallas import tpu as pltpu
from jax.experimental.pallas import tpu_sc as plsc

_L = 16
_NC = 2
_NS = 16
_NW = _NC * _NS


@functools.partial(jax.jit, static_argnums=(5,))
def _run(w2, s2, elast, nvec, fvec, S):
    R = w2.shape[0] // S
    NB = S + 1
    NO = 2 * NB
    RW = R // _NW
    SB = 128
    NSB = RW // SB
    GPB = SB // _L
    NBL = NB * _L

    mesh = plsc.VectorSubcoreMesh(core_axis_name="c", subcore_axis_name="s")

    @functools.partial(
        pl.kernel,
        out_type=jax.ShapeDtypeStruct((R * NO,), jnp.float32),
        mesh=mesh,
        compiler_params=pltpu.CompilerParams(needs_layout_passes=False),
        scratch_types=[
            pltpu.VMEM((SB * S,), jnp.float32),
            pltpu.VMEM((SB * S,), jnp.float32),
            pltpu.VMEM((SB * S,), jnp.float32),
            pltpu.VMEM((SB * S,), jnp.float32),
            pltpu.VMEM((SB,), jnp.float32),
            pltpu.VMEM((SB,), jnp.float32),
            pltpu.VMEM((SB,), jnp.float32),
            pltpu.VMEM((SB,), jnp.float32),
            pltpu.VMEM((SB,), jnp.float32),
            pltpu.VMEM((SB,), jnp.float32),
            pltpu.VMEM((SB * NO,), jnp.float32),
            pltpu.VMEM((SB * NO,), jnp.float32),
            pltpu.SemaphoreType.DMA,
            pltpu.SemaphoreType.DMA,
            pltpu.SemaphoreType.DMA,
            pltpu.SemaphoreType.DMA,
            pltpu.VMEM((NBL,), jnp.float32),
            pltpu.VMEM((NBL,), jnp.float32),
            pltpu.VMEM((NBL,), jnp.float32),
            pltpu.VMEM((NBL,), jnp.float32),
            pltpu.VMEM(((NB + 1) * _L,), jnp.int32),
            pltpu.VMEM(((NB + 1) * _L,), jnp.int32),
        ],
    )
    def body(w_hbm, s_hbm, e_hbm, n_hbm, f_hbm, out_hbm,
             wblk0, wblk1, sblk0, sblk1, eblk0, eblk1, nblk0, nblk1,
             fblk0, fblk1, outblk0, outblk1, isem0, isem1, osem0, osem1,
             cdf0, cdf1, eb0, eb1, h0, h1):
        cdfs, ebinss, hists = (cdf0, cdf1), (eb0, eb1), (h0, h1)
        bufs = ((wblk0, sblk0, eblk0, nblk0, fblk0), (wblk1, sblk1, eblk1, nblk1, fblk1))
        outblks = (outblk0, outblk1)
        isems, osems = (isem0, isem1), (osem0, osem1)
        wid = lax.axis_index("s") * _NC + lax.axis_index("c")
        base = wid * RW
        lane = lax.iota(jnp.int32, _L)
        ones_i = jnp.ones((_L,), jnp.int32)
        zeros_i = jnp.zeros((_L,), jnp.int32)
        zeros_f = jnp.zeros((_L,), jnp.float32)
        def make_gpair(wblk, sblk, eblk, nblk, fblk, outblk):
          def gpair(gp, _):
            st = []
            for t in (0, 1):
                g16 = (gp * 2 + t) * _L
                rows = g16 + lane
                near = nblk[pl.ds(g16, _L)]
                far = fblk[pl.ds(g16, _L)]
                st.append(dict(
                    g16=g16, rows=rows, rowsS=rows * S, rowsO=rows * NO,
                    near=near, fmn=far - near,
                    cdf=cdfs[t], ebins=ebinss[t], hist=hists[t],
                ))

            def pa(s, accs):
                out = []
                for t in (0, 1):
                    d = st[t]
                    ww = plsc.load_gather(wblk, [d["rowsS"] + s])
                    ee = plsc.load_gather(sblk, [d["rowsS"] + s])
                    acc = accs[t] + (ww + 0.01)
                    d["cdf"][pl.ds((s + 1) * _L, _L)] = acc
                    d["ebins"][pl.ds(s * _L, _L)] = ee
                    d["hist"][pl.ds(s * _L, _L)] = zeros_i
                    out.append(acc)
                return tuple(out)

            wss = plsc.parallel_loop(0, S, 1, unroll=4, carry=(zeros_f, zeros_f))(pa)
            pads = []
            for t in (0, 1):
                d = st[t]
                d["hist"][pl.ds(S * _L, _L)] = zeros_i
                d["hist"][pl.ds((S + 1) * _L, _L)] = zeros_i
                d["cdf"][pl.ds(0, _L)] = zeros_f
                d["ebins"][pl.ds(S * _L, _L)] = eblk[pl.ds(d["g16"], _L)]
                ws = wss[t]
                pad = jnp.maximum(0.0, 1e-5 - ws)
                pads.append((pad * (1.0 / S), 1.0 / (ws + pad)))

            def pb(k, carry):
                kf = k.astype(jnp.float32)
                for t in (0, 1):
                    d = st[t]
                    pad64, inv = pads[t]
                    cum = d["cdf"][pl.ds(k * _L, _L)]
                    c = jnp.minimum(1.0, (cum + kf * pad64) * inv)
                    d["cdf"][pl.ds(k * _L, _L)] = c
                    y = c * NB - 0.5
                    tr = y.astype(jnp.int32)
                    m = jnp.where(y > tr.astype(jnp.float32), tr + 1, tr)
                    plsc.addupdate_scatter(d["hist"], [m * _L + lane], ones_i)
                    ek = d["ebins"][pl.ds(k * _L, _L)]
                    val = d["near"] + ek * d["fmn"]
                    plsc.store_scatter(outblk, [d["rowsO"] + (m + k)], val)
                return carry

            plsc.parallel_loop(0, NB, 1, unroll=4, carry=jnp.int32(0))(pb)

            def pc(j, accs):
                u = (j.astype(jnp.float32) * 2.0 + 1.0) * (1.0 / NO)
                jp1 = j + 1
                out = []
                for t in (0, 1):
                    d = st[t]
                    acc = accs[t] + d["hist"][pl.ds(j * _L, _L)]
                    below = jnp.minimum(acc - 1, S)
                    above = jnp.minimum(acc, S)
                    bidx = below * _L + lane
                    aidx = above * _L + lane
                    c0 = plsc.load_gather(d["cdf"], [bidx])
                    c1 = plsc.load_gather(d["cdf"], [aidx])
                    e0 = plsc.load_gather(d["ebins"], [bidx])
                    e1 = plsc.load_gather(d["ebins"], [aidx])
                    num = u - c0
                    den = c1 - c0
                    tt = jnp.clip(num / den, 0.0, 1.0)
                    tt = jnp.where(den == 0.0, jnp.where(num > 0.0, 1.0, 0.0), tt)
                    bval = e0 + tt * (e1 - e0)
                    val = d["near"] + bval * d["fmn"]
                    plsc.store_scatter(outblk, [d["rowsO"] + (below + jp1)], val)
                    out.append(acc)
                return tuple(out)

            plsc.parallel_loop(0, NB, 1, unroll=4, carry=(zeros_i, zeros_i))(pc)
            return 0
          return gpair

        gpairs = tuple(make_gpair(*bufs[p], outblks[p]) for p in (0, 1))

        def start_in(p, sb):
            wblk, sblk, eblk, nblk, fblk = bufs[p]

            @pl.when(sb < NSB)
            def _():
                row0 = base + sb * SB
                pltpu.async_copy(w_hbm.at[pl.ds(row0 * S, SB * S)], wblk, isems[p])
                pltpu.async_copy(s_hbm.at[pl.ds(row0 * S, SB * S)], sblk, isems[p])
                pltpu.async_copy(e_hbm.at[pl.ds(row0, SB)], eblk, isems[p])
                pltpu.async_copy(n_hbm.at[pl.ds(row0, SB)], nblk, isems[p])
                pltpu.async_copy(f_hbm.at[pl.ds(row0, SB)], fblk, isems[p])

        def wait_in(p):
            wblk, sblk, eblk, nblk, fblk = bufs[p]
            pltpu.make_async_copy(w_hbm.at[pl.ds(0, SB * S)], wblk, isems[p]).wait()
            pltpu.make_async_copy(s_hbm.at[pl.ds(0, SB * S)], sblk, isems[p]).wait()
            pltpu.make_async_copy(e_hbm.at[pl.ds(0, SB)], eblk, isems[p]).wait()
            pltpu.make_async_copy(n_hbm.at[pl.ds(0, SB)], nblk, isems[p]).wait()
            pltpu.make_async_copy(f_hbm.at[pl.ds(0, SB)], fblk, isems[p]).wait()

        def wait_out(p):
            pltpu.make_async_copy(
                w_hbm.at[pl.ds(0, SB * NO)], outblks[p], osems[p]).wait()

        start_in(0, base * 0)

        def halfstep(h, _):
            for p in (0, 1):
                sb = h * 2 + p
                start_in(1 - p, sb + 1)
                wait_in(p)

                @pl.when(h > 0)
                def _():
                    wait_out(p)

                lax.fori_loop(0, GPB // 2, gpairs[p], 0)
                row0 = base + sb * SB
                pltpu.async_copy(
                    outblks[p], out_hbm.at[pl.ds(row0 * NO, SB * NO)], osems[p])
            return 0

        lax.fori_loop(0, NSB // 2, halfstep, 0)
        wait_out(0)
        wait_out(1)

    return body(w2, s2, elast, nvec, fvec).reshape(R, NO)


def kernel(weights, spacing_starts, spacing_ends, nears, fars, num_samples=64):
    R, S = weights.shape[0], weights.shape[1]
    w2 = weights.reshape(R * S)
    s2 = spacing_starts.reshape(R * S)
    elast = spacing_ends[:, -1, 0]
    return _run(w2, s2, elast, nears.reshape(R), fars.reshape(R), S)
